# baseline (speedup 1.0000x reference)
import functools

import numpy as np
import jax
import jax.numpy as jnp
from jax import lax
from jax.experimental import pallas as pl
from jax.experimental.pallas import tpu as pltpu
from jax.experimental.pallas import tpu_sc as plsc

_NA = 9
_PRE = 6000
_POST = 300
_THR = 0.7
_H = 64
_W = 64
_PIX = _H * _W
_TOT = _PIX * _NA
_ROWS = _TOT // 128
_MINI = np.int32(-2**31)


def _anchor_geom():
    ratios = np.array([0.5, 1.0, 2.0])
    scales = np.array([8.0, 16.0, 32.0])
    base = 16.0
    ctr = 0.5 * (base - 1.0)
    ws0 = np.round(np.sqrt(base * base / ratios))
    hs0 = np.round(ws0 * ratios)
    aw, ah = [], []
    for i in range(3):
        for s in scales:
            aw.append(ws0[i] * s)
            ah.append(hs0[i] * s)
    aw = np.array(aw, np.float32)
    ah = np.array(ah, np.float32)
    x1 = (ctr - 0.5 * (aw - 1.0)).astype(np.float32)
    y1 = (ctr - 0.5 * (ah - 1.0)).astype(np.float32)
    acx = (x1 + 0.5 * aw).astype(np.float32)
    acy = (y1 + 0.5 * ah).astype(np.float32)
    return np.stack([aw, ah, acx, acy]).astype(np.float32)


_AGEOM = _anchor_geom()


def _head_body(xpad_ref, w9_ref, cb_ref, cw_ref, cbb_ref, bw_ref, bbb_ref,
               geo_ref, ag_ref, sc_ref, x1_ref, y1_ref, x2_ref, y2_ref):
    acc = jnp.zeros((_PIX, 512), jnp.float32)
    for k in range(9):
        ky, kx = divmod(k, 3)
        patch = xpad_ref[ky:ky + _H, kx:kx + _W, :].reshape(_PIX, 256)
        acc = acc + jnp.dot(patch, w9_ref[k],
                            preferred_element_type=jnp.float32)
    rpn = jnp.maximum(acc + cb_ref[...], 0.0)

    cls = jnp.dot(rpn, cw_ref[...],
                  preferred_element_type=jnp.float32) + cbb_ref[...]
    s0 = cls[:, 0:_NA]
    s1 = cls[:, _NA:2 * _NA]
    mx = jnp.maximum(s0, s1)
    e0 = jnp.exp(s0 - mx)
    e1 = jnp.exp(s1 - mx)
    scores = e1 / (e0 + e1)

    boxd = jnp.dot(rpn, bw_ref[...],
                   preferred_element_type=jnp.float32) + bbb_ref[...]
    dx = boxd[:, 0:9]
    dy = boxd[:, 9:18]
    dw = boxd[:, 18:27]
    dh = boxd[:, 27:36]

    aw = ag_ref[0:1, :]
    ah = ag_ref[1:2, :]
    pix = jax.lax.broadcasted_iota(jnp.int32, (_PIX, _NA), 0)
    px = ((pix % _W) * 16).astype(jnp.float32)
    py = ((pix // _W) * 16).astype(jnp.float32)
    cx = ag_ref[2:3, :] + px
    cy = ag_ref[3:4, :] + py

    pcx = dx * aw + cx
    pcy = dy * ah + cy
    pw = jnp.exp(dw) * aw
    ph = jnp.exp(dh) * ah
    x1 = pcx - 0.5 * pw
    y1 = pcy - 0.5 * ph
    x2 = pcx + 0.5 * pw
    y2 = pcy + 0.5 * ph

    Hm1 = geo_ref[0:1, 0:1] - 1.0
    Wm1 = geo_ref[0:1, 1:2] - 1.0
    msz = 16.0 * geo_ref[0:1, 2:3]
    x1c = jnp.clip(x1, 0.0, Wm1)
    y1c = jnp.clip(y1, 0.0, Hm1)
    x2c = jnp.clip(x2, 0.0, Wm1)
    y2c = jnp.clip(y2, 0.0, Hm1)

    ws = x2c - x1c + 1.0
    hs = y2c - y1c + 1.0
    valid = (ws >= msz) & (hs >= msz)
    sc_ref[...] = jnp.where(valid, scores, -1e9)
    x1_ref[...] = x1c
    y1_ref[...] = y1c
    x2_ref[...] = x2c
    y2_ref[...] = y2c


_NWK = 16
_CHUNK = _TOT // _NWK
_CAP = 6144
_CROWS = _CAP // 128
_SLAB = _CAP // _NWK
_SHW = _CAP + 16


def _thresh_body(sc_ref, sm_ref, base_ref):
    s = sc_ref[...]
    bits = jax.lax.bitcast_convert_type(s, jnp.int32)
    key = bits ^ ((bits >> 31) & np.int32(0x7FFFFFFF))
    tu = jnp.int32(0)
    for b in range(31, -1, -1):
        bit = _MINI if b == 31 else np.int32(1 << b)
        cand = tu | bit
        cnt = jnp.sum((key >= (cand ^ _MINI)).astype(jnp.int32))
        tu = jnp.where(cnt >= _PRE, cand, tu)
    kt = tu ^ _MINI
    mask = key >= kt
    sm_ref[...] = jnp.where(mask, s, -jnp.inf)
    mi = mask.astype(jnp.int32)
    rit = jax.lax.broadcasted_iota(jnp.int32, (_ROWS, 128), 0)
    l16 = jax.lax.broadcasted_iota(jnp.int32, (1, 16), 1)
    bases = jnp.zeros((1, 16), jnp.int32)
    rows_per_chunk = _ROWS // _NWK
    for w in range(_NWK):
        bw = jnp.sum(jnp.where(rit < w * rows_per_chunk, mi, 0))
        bases = jnp.where(l16 == w, bw, bases)
    base_ref[...] = bases


def _compact_body(sm_hbm, x1_hbm, y1_hbm, x2_hbm, y2_hbm, bases_hbm,
                  os_hbm, ox1_hbm, oy1_hbm, ox2_hbm, oy2_hbm,
                  s_v, x1_v, y1_v, x2_v, y2_v, bas_v, idx_v, fil_v,
                  sh_s, sh_x1, sh_y1, sh_x2, sh_y2, sem):
    wid = lax.axis_index("s")
    ebase = wid * _CHUNK

    pltpu.sync_copy(sm_hbm.at[pl.ds(ebase, _CHUNK)], s_v)
    pltpu.sync_copy(x1_hbm.at[pl.ds(ebase, _CHUNK)], x1_v)
    pltpu.sync_copy(y1_hbm.at[pl.ds(ebase, _CHUNK)], y1_v)
    pltpu.sync_copy(x2_hbm.at[pl.ds(ebase, _CHUNK)], x2_v)
    pltpu.sync_copy(y2_hbm.at[pl.ds(ebase, _CHUNK)], y2_v)
    pltpu.sync_copy(bases_hbm, bas_v)

    ninf = jnp.full((16,), -jnp.inf, jnp.float32)
    zero = jnp.full((16,), 0.0, jnp.float32)
    for i in range(_SLAB // 16):
        fil_v[pl.ds(i * 16, 16)] = ninf
    pltpu.sync_copy(fil_v, sh_s.at[pl.ds(wid * _SLAB, _SLAB)])
    for i in range(_SLAB // 16):
        fil_v[pl.ds(i * 16, 16)] = zero
    pltpu.sync_copy(fil_v, sh_x1.at[pl.ds(wid * _SLAB, _SLAB)])
    pltpu.sync_copy(fil_v, sh_y1.at[pl.ds(wid * _SLAB, _SLAB)])
    pltpu.sync_copy(fil_v, sh_x2.at[pl.ds(wid * _SLAB, _SLAB)])
    pltpu.sync_copy(fil_v, sh_y2.at[pl.ds(wid * _SLAB, _SLAB)])

    li = lax.iota(jnp.int32, 16)
    mybase = jnp.sum(jnp.where(li == wid, bas_v[...], 0))

    ngrp = _CHUNK // 128

    def build(g, off):
        for j in range(8):
            k = g * 8 + j
            v = s_v[pl.ds(k * 16, 16)]
            m = v > -jnp.inf
            mi = m.astype(jnp.int32)
            inc = plsc.cumsum(mi)
            pos = jnp.broadcast_to(off, (16,)) + (inc - mi)
            pos = jnp.where(m & (pos < _CAP), pos, _CAP)
            idx_v[g, pl.ds(j * 16, 16)] = pos
            off = off + jnp.sum(mi)
        return off

    lax.fori_loop(0, ngrp, build, mybase)

    def scatter(g, carry):
        cps = [
            pltpu.make_async_copy(s_v.at[pl.ds(g * 128, 128)],
                                  sh_s.at[idx_v.at[g]], sem),
            pltpu.make_async_copy(x1_v.at[pl.ds(g * 128, 128)],
                                  sh_x1.at[idx_v.at[g]], sem),
            pltpu.make_async_copy(y1_v.at[pl.ds(g * 128, 128)],
                                  sh_y1.at[idx_v.at[g]], sem),
            pltpu.make_async_copy(x2_v.at[pl.ds(g * 128, 128)],
                                  sh_x2.at[idx_v.at[g]], sem),
            pltpu.make_async_copy(y2_v.at[pl.ds(g * 128, 128)],
                                  sh_y2.at[idx_v.at[g]], sem),
        ]
        for cp in cps:
            cp.start()
        for cp in cps:
            cp.wait()
        return carry

    lax.fori_loop(0, ngrp, scatter, jnp.int32(0))

    plsc.subcore_barrier()

    sl = pl.ds(wid * _SLAB, _SLAB)
    pltpu.sync_copy(sh_s.at[sl], os_hbm.at[sl])
    pltpu.sync_copy(sh_x1.at[sl], ox1_hbm.at[sl])
    pltpu.sync_copy(sh_y1.at[sl], oy1_hbm.at[sl])
    pltpu.sync_copy(sh_x2.at[sl], ox2_hbm.at[sl])
    pltpu.sync_copy(sh_y2.at[sl], oy2_hbm.at[sl])


def _nms_body(sc_ref, x1_ref, y1_ref, x2_ref, y2_ref, out_ref):
    s = sc_ref[...]
    x1 = x1_ref[...]
    y1 = y1_ref[...]
    x2 = x2_ref[...]
    y2 = y2_ref[...]

    areas = (x2 - x1 + 1.0) * (y2 - y1 + 1.0)
    xp2 = x2 + 1.0
    yp2 = y2 + 1.0
    fidx = (jax.lax.broadcasted_iota(jnp.int32, (_CROWS, 128), 0) * 128
            + jax.lax.broadcasted_iota(jnp.int32, (_CROWS, 128), 1))
    l128 = jax.lax.broadcasted_iota(jnp.int32, (1, 128), 1)
    l8 = jax.lax.broadcasted_iota(jnp.int32, (1, 8), 1)

    def body(i, carry):
        s, j0 = carry
        m = jnp.max(s)
        j = jnp.min(jnp.where(s == m, fidx, jnp.int32(1 << 30)))
        j = jnp.where(m == -jnp.inf, j0, j)
        j0 = jnp.where(i == 0, j, j0)
        row = j // 128
        lane = j % 128
        lsel = l128 == lane

        def ext(ref):
            return jnp.sum(jnp.where(lsel, ref[pl.ds(row, 1), :], 0.0))

        bx1 = ext(x1_ref)
        by1 = ext(y1_ref)
        bx2 = ext(x2_ref)
        by2 = ext(y2_ref)
        bar = (bx2 - bx1 + 1.0) * (by2 - by1 + 1.0)
        iw = jnp.maximum(0.0, jnp.minimum(bx2 + 1.0, xp2)
                         - jnp.maximum(bx1, x1))
        ih = jnp.maximum(0.0, jnp.minimum(by2 + 1.0, yp2)
                         - jnp.maximum(by1, y1))
        inter = iw * ih
        den = (areas + bar) - inter
        s = jnp.where(inter > _THR * den, -jnp.inf, s)
        nr = jnp.where(l8 == 1, bx1,
                       jnp.where(l8 == 2, by1,
                                 jnp.where(l8 == 3, bx2,
                                           jnp.where(l8 == 4, by2, 0.0))))
        out_ref[pl.ds(i, 1), :] = nr
        return s, j0

    jax.lax.fori_loop(0, _POST, body, (s, jnp.int32(0)))


def _run_head(xpad, w9, cb, cw, cbb, bw, bbb, geo, interpret=False):
    shp = jax.ShapeDtypeStruct((_PIX, _NA), jnp.float32)
    return pl.pallas_call(
        _head_body,
        out_shape=[shp] * 5,
        interpret=interpret,
    )(xpad, w9, cb, cw, cbb, bw, bbb, geo, jnp.asarray(_AGEOM))


def _run_thresh(sc, interpret=False):
    return pl.pallas_call(
        _thresh_body,
        out_shape=[jax.ShapeDtypeStruct((_ROWS, 128), jnp.float32),
                   jax.ShapeDtypeStruct((1, 16), jnp.int32)],
        interpret=interpret,
    )(sc)


def _run_compact(sm, x1, y1, x2, y2, bases):
    mesh = plsc.VectorSubcoreMesh(core_axis_name="c", subcore_axis_name="s",
                                  num_cores=1)
    f32 = jnp.float32
    kern = functools.partial(
        pl.kernel,
        mesh=mesh,
        compiler_params=pltpu.CompilerParams(needs_layout_passes=False),
        out_type=[jax.ShapeDtypeStruct((_CAP,), f32)] * 5,
        scratch_types=[
            pltpu.VMEM((_CHUNK,), f32),
            pltpu.VMEM((_CHUNK,), f32),
            pltpu.VMEM((_CHUNK,), f32),
            pltpu.VMEM((_CHUNK,), f32),
            pltpu.VMEM((_CHUNK,), f32),
            pltpu.VMEM((16,), jnp.int32),
            pltpu.VMEM((_CHUNK // 128, 128), jnp.int32),
            pltpu.VMEM((_SLAB,), f32),
            pltpu.VMEM_SHARED((_SHW,), f32),
            pltpu.VMEM_SHARED((_SHW,), f32),
            pltpu.VMEM_SHARED((_SHW,), f32),
            pltpu.VMEM_SHARED((_SHW,), f32),
            pltpu.VMEM_SHARED((_SHW,), f32),
            pltpu.SemaphoreType.DMA,
        ],
    )(_compact_body)
    return kern(sm, x1, y1, x2, y2, bases)


def _run_nms(sc, x1, y1, x2, y2, interpret=False):
    return pl.pallas_call(
        _nms_body,
        out_shape=jax.ShapeDtypeStruct((304, 8), jnp.float32),
        interpret=interpret,
    )(sc, x1, y1, x2, y2)


def _kernel_impl(features, gt_boxes, im_info, conv_w, conv_b, cls_w, cls_b,
                 box_w, box_b, interpret=False):
    x = features[0].transpose(1, 2, 0)
    xpad = jnp.pad(x, ((1, 1), (1, 1), (0, 0)))
    w9 = conv_w.transpose(2, 3, 1, 0).reshape(9, 256, 512)
    cb = conv_b.reshape(1, 512)
    cw = cls_w[:, :, 0, 0].T
    cbb = cls_b.reshape(1, 18)
    perm = np.array([a * 4 + d for d in range(4) for a in range(_NA)])
    bw = box_w[:, :, 0, 0].T[:, perm]
    bbb = box_b[perm].reshape(1, 36)
    geo = jnp.pad(im_info, ((0, 0), (0, 125)))

    sc, x1, y1, x2, y2 = _run_head(xpad, w9, cb, cw, cbb, bw, bbb, geo,
                                   interpret=interpret)

    def _r(t):
        return t.reshape(_TOT).reshape(_ROWS, 128)

    sm, bases = _run_thresh(_r(sc), interpret=interpret)
    cs, cx1, cy1, cx2, cy2 = _run_compact(
        sm.reshape(_TOT), _r(x1).reshape(_TOT), _r(y1).reshape(_TOT),
        _r(x2).reshape(_TOT), _r(y2).reshape(_TOT), bases.reshape(16))

    def _c(t):
        return t.reshape(_CROWS, 128)

    out = _run_nms(_c(cs), _c(cx1), _c(cy1), _c(cx2), _c(cy2),
                   interpret=interpret)
    return out[:_POST, :5]


def kernel(features, gt_boxes, im_info, conv_w, conv_b, cls_w, cls_b,
           box_w, box_b):
    return _kernel_impl(features, gt_boxes, im_info, conv_w, conv_b,
                        cls_w, cls_b, box_w, box_b)


# speedup vs baseline: 1.2864x; 1.2864x over previous
import functools

import numpy as np
import jax
import jax.numpy as jnp
from jax import lax
from jax.experimental import pallas as pl
from jax.experimental.pallas import tpu as pltpu
from jax.experimental.pallas import tpu_sc as plsc

_NA = 9
_PRE = 6000
_POST = 300
_THR = 0.7
_H = 64
_W = 64
_PIX = _H * _W
_TOT = _PIX * _NA
_ROWS = _TOT // 128
_MINI = np.int32(-2**31)


def _anchor_geom():
    ratios = np.array([0.5, 1.0, 2.0])
    scales = np.array([8.0, 16.0, 32.0])
    base = 16.0
    ctr = 0.5 * (base - 1.0)
    ws0 = np.round(np.sqrt(base * base / ratios))
    hs0 = np.round(ws0 * ratios)
    aw, ah = [], []
    for i in range(3):
        for s in scales:
            aw.append(ws0[i] * s)
            ah.append(hs0[i] * s)
    aw = np.array(aw, np.float32)
    ah = np.array(ah, np.float32)
    x1 = (ctr - 0.5 * (aw - 1.0)).astype(np.float32)
    y1 = (ctr - 0.5 * (ah - 1.0)).astype(np.float32)
    acx = (x1 + 0.5 * aw).astype(np.float32)
    acy = (y1 + 0.5 * ah).astype(np.float32)
    return np.stack([aw, ah, acx, acy]).astype(np.float32)


_AGEOM = _anchor_geom()


def _head_body(xpad_ref, w9_ref, cb_ref, cw_ref, cbb_ref, bw_ref, bbb_ref,
               geo_ref, ag_ref, sc_ref, x1_ref, y1_ref, x2_ref, y2_ref):
    acc = jnp.zeros((_PIX, 512), jnp.float32)
    for k in range(9):
        ky, kx = divmod(k, 3)
        patch = xpad_ref[ky:ky + _H, kx:kx + _W, :].reshape(_PIX, 256)
        acc = acc + jnp.dot(patch, w9_ref[k],
                            preferred_element_type=jnp.float32)
    rpn = jnp.maximum(acc + cb_ref[...], 0.0)

    cls = jnp.dot(rpn, cw_ref[...],
                  preferred_element_type=jnp.float32) + cbb_ref[...]
    s0 = cls[:, 0:_NA]
    s1 = cls[:, _NA:2 * _NA]
    mx = jnp.maximum(s0, s1)
    e0 = jnp.exp(s0 - mx)
    e1 = jnp.exp(s1 - mx)
    scores = e1 / (e0 + e1)

    boxd = jnp.dot(rpn, bw_ref[...],
                   preferred_element_type=jnp.float32) + bbb_ref[...]
    dx = boxd[:, 0:9]
    dy = boxd[:, 9:18]
    dw = boxd[:, 18:27]
    dh = boxd[:, 27:36]

    aw = ag_ref[0:1, :]
    ah = ag_ref[1:2, :]
    pix = jax.lax.broadcasted_iota(jnp.int32, (_PIX, _NA), 0)
    px = ((pix % _W) * 16).astype(jnp.float32)
    py = ((pix // _W) * 16).astype(jnp.float32)
    cx = ag_ref[2:3, :] + px
    cy = ag_ref[3:4, :] + py

    pcx = dx * aw + cx
    pcy = dy * ah + cy
    pw = jnp.exp(dw) * aw
    ph = jnp.exp(dh) * ah
    x1 = pcx - 0.5 * pw
    y1 = pcy - 0.5 * ph
    x2 = pcx + 0.5 * pw
    y2 = pcy + 0.5 * ph

    Hm1 = geo_ref[0:1, 0:1] - 1.0
    Wm1 = geo_ref[0:1, 1:2] - 1.0
    msz = 16.0 * geo_ref[0:1, 2:3]
    x1c = jnp.clip(x1, 0.0, Wm1)
    y1c = jnp.clip(y1, 0.0, Hm1)
    x2c = jnp.clip(x2, 0.0, Wm1)
    y2c = jnp.clip(y2, 0.0, Hm1)

    ws = x2c - x1c + 1.0
    hs = y2c - y1c + 1.0
    valid = (ws >= msz) & (hs >= msz)
    sc_ref[...] = jnp.where(valid, scores, -1e9)
    x1_ref[...] = x1c
    y1_ref[...] = y1c
    x2_ref[...] = x2c
    y2_ref[...] = y2c


_NWK = 16
_CHUNK = _TOT // _NWK
_CAP = 6144
_CROWS = _CAP // 128
_SLAB = _CAP // _NWK
_SHW = _CAP + 16


def _thresh_body(sc_ref, sm_ref, base_ref):
    s = sc_ref[...]
    bits = jax.lax.bitcast_convert_type(s, jnp.int32)
    key = bits ^ ((bits >> 31) & np.int32(0x7FFFFFFF))
    tu = jnp.int32(0)
    for b in range(31, -1, -1):
        bit = _MINI if b == 31 else np.int32(1 << b)
        cand = tu | bit
        cnt = jnp.sum((key >= (cand ^ _MINI)).astype(jnp.int32))
        tu = jnp.where(cnt >= _PRE, cand, tu)
    kt = tu ^ _MINI
    mask = key >= kt
    sm_ref[...] = jnp.where(mask, s, -jnp.inf)
    mi = mask.astype(jnp.int32)
    rit = jax.lax.broadcasted_iota(jnp.int32, (_ROWS, 128), 0)
    l16 = jax.lax.broadcasted_iota(jnp.int32, (1, 16), 1)
    bases = jnp.zeros((1, 16), jnp.int32)
    rows_per_chunk = _ROWS // _NWK
    for w in range(_NWK):
        bw = jnp.sum(jnp.where(rit < w * rows_per_chunk, mi, 0))
        bases = jnp.where(l16 == w, bw, bases)
    base_ref[...] = bases


def _compact_body(sm_hbm, x1_hbm, y1_hbm, x2_hbm, y2_hbm, bases_hbm,
                  os_hbm, ox1_hbm, oy1_hbm, ox2_hbm, oy2_hbm,
                  s_v, x1_v, y1_v, x2_v, y2_v, bas_v, idx_v, fil_v,
                  sh_s, sh_x1, sh_y1, sh_x2, sh_y2, sem):
    wid = lax.axis_index("s")
    ebase = wid * _CHUNK

    pltpu.sync_copy(sm_hbm.at[pl.ds(ebase, _CHUNK)], s_v)
    pltpu.sync_copy(x1_hbm.at[pl.ds(ebase, _CHUNK)], x1_v)
    pltpu.sync_copy(y1_hbm.at[pl.ds(ebase, _CHUNK)], y1_v)
    pltpu.sync_copy(x2_hbm.at[pl.ds(ebase, _CHUNK)], x2_v)
    pltpu.sync_copy(y2_hbm.at[pl.ds(ebase, _CHUNK)], y2_v)
    pltpu.sync_copy(bases_hbm, bas_v)

    ninf = jnp.full((16,), -jnp.inf, jnp.float32)
    zero = jnp.full((16,), 0.0, jnp.float32)
    for i in range(_SLAB // 16):
        fil_v[pl.ds(i * 16, 16)] = ninf
    pltpu.sync_copy(fil_v, sh_s.at[pl.ds(wid * _SLAB, _SLAB)])
    for i in range(_SLAB // 16):
        fil_v[pl.ds(i * 16, 16)] = zero
    pltpu.sync_copy(fil_v, sh_x1.at[pl.ds(wid * _SLAB, _SLAB)])
    pltpu.sync_copy(fil_v, sh_y1.at[pl.ds(wid * _SLAB, _SLAB)])
    pltpu.sync_copy(fil_v, sh_x2.at[pl.ds(wid * _SLAB, _SLAB)])
    pltpu.sync_copy(fil_v, sh_y2.at[pl.ds(wid * _SLAB, _SLAB)])

    li = lax.iota(jnp.int32, 16)
    mybase = jnp.sum(jnp.where(li == wid, bas_v[...], 0))

    ngrp = _CHUNK // 128

    def build(g, off):
        for j in range(8):
            k = g * 8 + j
            v = s_v[pl.ds(k * 16, 16)]
            m = v > -jnp.inf
            mi = m.astype(jnp.int32)
            inc = plsc.cumsum(mi)
            pos = jnp.broadcast_to(off, (16,)) + (inc - mi)
            pos = jnp.where(m & (pos < _CAP), pos, _CAP)
            idx_v[g, pl.ds(j * 16, 16)] = pos
            off = off + jnp.sum(mi)
        return off

    lax.fori_loop(0, ngrp, build, mybase)

    def scatter(g, carry):
        cps = [
            pltpu.make_async_copy(s_v.at[pl.ds(g * 128, 128)],
                                  sh_s.at[idx_v.at[g]], sem),
            pltpu.make_async_copy(x1_v.at[pl.ds(g * 128, 128)],
                                  sh_x1.at[idx_v.at[g]], sem),
            pltpu.make_async_copy(y1_v.at[pl.ds(g * 128, 128)],
                                  sh_y1.at[idx_v.at[g]], sem),
            pltpu.make_async_copy(x2_v.at[pl.ds(g * 128, 128)],
                                  sh_x2.at[idx_v.at[g]], sem),
            pltpu.make_async_copy(y2_v.at[pl.ds(g * 128, 128)],
                                  sh_y2.at[idx_v.at[g]], sem),
        ]
        for cp in cps:
            cp.start()
        for cp in cps:
            cp.wait()
        return carry

    lax.fori_loop(0, ngrp, scatter, jnp.int32(0))

    plsc.subcore_barrier()

    sl = pl.ds(wid * _SLAB, _SLAB)
    pltpu.sync_copy(sh_s.at[sl], os_hbm.at[sl])
    pltpu.sync_copy(sh_x1.at[sl], ox1_hbm.at[sl])
    pltpu.sync_copy(sh_y1.at[sl], oy1_hbm.at[sl])
    pltpu.sync_copy(sh_x2.at[sl], ox2_hbm.at[sl])
    pltpu.sync_copy(sh_y2.at[sl], oy2_hbm.at[sl])


def _nms_body(sc_ref, x1_ref, y1_ref, x2_ref, y2_ref, out_ref):
    s = sc_ref[...]
    x1 = x1_ref[...]
    y1 = y1_ref[...]
    x2 = x2_ref[...]
    y2 = y2_ref[...]

    areas = (x2 - x1 + 1.0) * (y2 - y1 + 1.0)
    xp2 = x2 + 1.0
    yp2 = y2 + 1.0
    l8 = jax.lax.broadcasted_iota(jnp.int32, (1, 8), 1)

    def body(i, carry):
        s, fx1, fy1, fx2, fy2 = carry
        m = jnp.max(s)
        neg = m == -jnp.inf
        sel = s == m
        bx1 = jnp.where(neg, fx1, jnp.sum(jnp.where(sel, x1, 0.0)))
        by1 = jnp.where(neg, fy1, jnp.sum(jnp.where(sel, y1, 0.0)))
        bx2 = jnp.where(neg, fx2, jnp.sum(jnp.where(sel, x2, 0.0)))
        by2 = jnp.where(neg, fy2, jnp.sum(jnp.where(sel, y2, 0.0)))
        first = i == 0
        fx1 = jnp.where(first, bx1, fx1)
        fy1 = jnp.where(first, by1, fy1)
        fx2 = jnp.where(first, bx2, fx2)
        fy2 = jnp.where(first, by2, fy2)
        bar = (bx2 - bx1 + 1.0) * (by2 - by1 + 1.0)
        iw = jnp.maximum(0.0, jnp.minimum(bx2 + 1.0, xp2)
                         - jnp.maximum(bx1, x1))
        ih = jnp.maximum(0.0, jnp.minimum(by2 + 1.0, yp2)
                         - jnp.maximum(by1, y1))
        inter = iw * ih
        den = (areas + bar) - inter
        s = jnp.where(inter > _THR * den, -jnp.inf, s)
        nr = jnp.where(l8 == 1, bx1,
                       jnp.where(l8 == 2, by1,
                                 jnp.where(l8 == 3, bx2,
                                           jnp.where(l8 == 4, by2, 0.0))))
        out_ref[pl.ds(i, 1), :] = nr
        return s, fx1, fy1, fx2, fy2

    z = jnp.float32(0.0)
    jax.lax.fori_loop(0, _POST, body, (s, z, z, z, z))


def _run_head(xpad, w9, cb, cw, cbb, bw, bbb, geo, interpret=False):
    shp = jax.ShapeDtypeStruct((_PIX, _NA), jnp.float32)
    return pl.pallas_call(
        _head_body,
        out_shape=[shp] * 5,
        interpret=interpret,
    )(xpad, w9, cb, cw, cbb, bw, bbb, geo, jnp.asarray(_AGEOM))


def _run_thresh(sc, interpret=False):
    return pl.pallas_call(
        _thresh_body,
        out_shape=[jax.ShapeDtypeStruct((_ROWS, 128), jnp.float32),
                   jax.ShapeDtypeStruct((1, 16), jnp.int32)],
        interpret=interpret,
    )(sc)


def _run_compact(sm, x1, y1, x2, y2, bases):
    mesh = plsc.VectorSubcoreMesh(core_axis_name="c", subcore_axis_name="s",
                                  num_cores=1)
    f32 = jnp.float32
    kern = functools.partial(
        pl.kernel,
        mesh=mesh,
        compiler_params=pltpu.CompilerParams(needs_layout_passes=False),
        out_type=[jax.ShapeDtypeStruct((_CAP,), f32)] * 5,
        scratch_types=[
            pltpu.VMEM((_CHUNK,), f32),
            pltpu.VMEM((_CHUNK,), f32),
            pltpu.VMEM((_CHUNK,), f32),
            pltpu.VMEM((_CHUNK,), f32),
            pltpu.VMEM((_CHUNK,), f32),
            pltpu.VMEM((16,), jnp.int32),
            pltpu.VMEM((_CHUNK // 128, 128), jnp.int32),
            pltpu.VMEM((_SLAB,), f32),
            pltpu.VMEM_SHARED((_SHW,), f32),
            pltpu.VMEM_SHARED((_SHW,), f32),
            pltpu.VMEM_SHARED((_SHW,), f32),
            pltpu.VMEM_SHARED((_SHW,), f32),
            pltpu.VMEM_SHARED((_SHW,), f32),
            pltpu.SemaphoreType.DMA,
        ],
    )(_compact_body)
    return kern(sm, x1, y1, x2, y2, bases)


def _run_nms(sc, x1, y1, x2, y2, interpret=False):
    return pl.pallas_call(
        _nms_body,
        out_shape=jax.ShapeDtypeStruct((304, 8), jnp.float32),
        interpret=interpret,
    )(sc, x1, y1, x2, y2)


def _kernel_impl(features, gt_boxes, im_info, conv_w, conv_b, cls_w, cls_b,
                 box_w, box_b, interpret=False):
    x = features[0].transpose(1, 2, 0)
    xpad = jnp.pad(x, ((1, 1), (1, 1), (0, 0)))
    w9 = conv_w.transpose(2, 3, 1, 0).reshape(9, 256, 512)
    cb = conv_b.reshape(1, 512)
    cw = cls_w[:, :, 0, 0].T
    cbb = cls_b.reshape(1, 18)
    perm = np.array([a * 4 + d for d in range(4) for a in range(_NA)])
    bw = box_w[:, :, 0, 0].T[:, perm]
    bbb = box_b[perm].reshape(1, 36)
    geo = jnp.pad(im_info, ((0, 0), (0, 125)))

    sc, x1, y1, x2, y2 = _run_head(xpad, w9, cb, cw, cbb, bw, bbb, geo,
                                   interpret=interpret)

    def _r(t):
        return t.reshape(_TOT).reshape(_ROWS, 128)

    sm, bases = _run_thresh(_r(sc), interpret=interpret)
    cs, cx1, cy1, cx2, cy2 = _run_compact(
        sm.reshape(_TOT), _r(x1).reshape(_TOT), _r(y1).reshape(_TOT),
        _r(x2).reshape(_TOT), _r(y2).reshape(_TOT), bases.reshape(16))

    def _c(t):
        return t.reshape(_CROWS, 128)

    out = _run_nms(_c(cs), _c(cx1), _c(cy1), _c(cx2), _c(cy2),
                   interpret=interpret)
    return out[:_POST, :5]


def kernel(features, gt_boxes, im_info, conv_w, conv_b, cls_w, cls_b,
           box_w, box_b):
    return _kernel_impl(features, gt_boxes, im_info, conv_w, conv_b,
                        cls_w, cls_b, box_w, box_b)
